# baseline (speedup 1.0000x reference)
import functools

import jax
import jax.numpy as jnp
from jax import lax
from jax.experimental import pallas as pl
from jax.experimental.pallas import tpu as pltpu

_T_BLK = 128
_CAPACITY = 0.125


def _router_kernel(scal_ref, a_ref, p_ref, g_ref, bin_ref, dst_scr, dch_scr,
                   *, B, T, D, k, nt):
    t = pl.program_id(0)
    a = a_ref[...]
    p = p_ref[...]
    dst = jnp.sum(a * a, axis=-1) / D
    d = a - p
    dch = jnp.sum(d * d, axis=-1) / D
    dst_scr[:, pl.ds(t * _T_BLK, _T_BLK)] = dst
    dch_scr[:, pl.ds(t * _T_BLK, _T_BLK)] = dch

    @pl.when(t == nt - 1)
    def _phase2():
        H = T // 2
        dst_all = dst_scr[...].reshape(2 * B, H)
        dch_all = dch_scr[...].reshape(2 * B, H)
        log_oce = scal_ref[0]
        m_cu = scal_ref[1]
        bce = scal_ref[2]
        bcu = scal_ref[3]
        ma = jnp.sum(dst_all) / (B * T)
        ce = dst_all - (dch_all - log_oce)
        cu = dst_all - m_cu * ma
        s_ce = jax.nn.sigmoid(bce * ce)
        s_cu = jax.nn.sigmoid(bcu * cu)
        g = s_ce + s_cu - s_ce * s_cu
        g_ref[...] = g.reshape(B, T)

        u = lax.bitcast_convert_type(g, jnp.int32)
        idx = (lax.broadcasted_iota(jnp.int32, (2 * B, H), 1)
               + (lax.broadcasted_iota(jnp.int32, (2 * B, H), 0) % 2) * H)

        ri = lax.broadcasted_iota(jnp.int32, (2 * B, 2 * B), 0)
        ci = lax.broadcasted_iota(jnp.int32, (2 * B, 2 * B), 1)
        pmat = ((ri // 2) == (ci // 2)).astype(jnp.float32)

        def count8(mask8):
            cnt = jnp.sum(mask8.astype(jnp.float32), axis=1, keepdims=True)
            return jax.lax.dot(pmat, cnt,
                               precision=jax.lax.Precision.HIGHEST)

        def val_bit(i, cand):
            trial = cand | (jnp.int32(1) << (jnp.int32(29) - i))
            cnt = count8(u >= trial)
            return jnp.where(cnt >= k, trial, cand)

        thr = lax.fori_loop(0, 30, val_bit, jnp.zeros((2 * B, 1), jnp.int32))
        n_gt = count8(u > thr)
        need = k - n_gt
        tie = u == thr

        def idx_bit(i, ic):
            trial = ic | (jnp.int32(1) << (jnp.int32(12) - i))
            cnt = count8(tie & (idx < trial))
            return jnp.where(cnt < need, trial, ic)

        xthr = lax.fori_loop(0, 13, idx_bit, jnp.zeros((2 * B, 1), jnp.int32))
        mask = (u > thr) | (tie & (idx <= xthr))
        bin_ref[...] = mask.astype(jnp.float32).reshape(B, T)


def kernel(actual_residual, predicted_residual, o_ce, m_cu, beta_ce, beta_cu):
    B, T, D = actual_residual.shape
    k = max(1, int(T * _CAPACITY))
    nt = T // _T_BLK
    scal = jnp.stack([
        jnp.log(o_ce + 1e-10),
        m_cu,
        jax.nn.softplus(beta_ce),
        jax.nn.softplus(beta_cu),
    ]).astype(jnp.float32)

    body = functools.partial(_router_kernel, B=B, T=T, D=D, k=k, nt=nt)
    g, binary = pl.pallas_call(
        body,
        grid=(nt,),
        in_specs=[
            pl.BlockSpec(memory_space=pltpu.SMEM),
            pl.BlockSpec((B, _T_BLK, D), lambda t: (0, t, 0)),
            pl.BlockSpec((B, _T_BLK, D), lambda t: (0, t, 0)),
        ],
        out_specs=[
            pl.BlockSpec((B, T), lambda t: (0, 0)),
            pl.BlockSpec((B, T), lambda t: (0, 0)),
        ],
        out_shape=[
            jax.ShapeDtypeStruct((B, T), jnp.float32),
            jax.ShapeDtypeStruct((B, T), jnp.float32),
        ],
        scratch_shapes=[
            pltpu.VMEM((B, T), jnp.float32),
            pltpu.VMEM((B, T), jnp.float32),
        ],
        compiler_params=pltpu.CompilerParams(
            dimension_semantics=("arbitrary",),
        ),
    )(scal, actual_residual, predicted_residual)
    return (g, binary)
